# baseline (speedup 1.0000x reference)
import functools

import jax
import jax.numpy as jnp
from jax.experimental import pallas as pl
from jax.experimental.pallas import tpu as pltpu

_TEMPERATURE = 2.2
_TINY = float(jnp.finfo(jnp.float32).tiny)
_ROT_A = (13, 15, 26, 6)
_ROT_B = (17, 29, 16, 24)


def _threefry_bits(flat_u32):
    ks = (jnp.uint32(0), jnp.uint32(42), jnp.uint32(0x1BD11BDA ^ 42))
    x0 = jnp.zeros_like(flat_u32)
    x1 = flat_u32 + ks[1]
    for i in range(5):
        rots = _ROT_A if i % 2 == 0 else _ROT_B
        for r in rots:
            x0 = x0 + x1
            x1 = ((x1 << jnp.uint32(r)) | (x1 >> jnp.uint32(32 - r))) ^ x0
        x0 = x0 + ks[(i + 1) % 3]
        x1 = x1 + ks[(i + 2) % 3] + jnp.uint32(i + 1)
    return x0 ^ x1


def _scan_body(x_ref, o_ref, val_ref, idx_ref, *, vocab, block_cols):
    pid = pl.program_id(0)
    nblk = pl.num_programs(0)
    blk = x_ref[...]
    rows, cw = blk.shape

    col = jax.lax.broadcasted_iota(jnp.int32, (rows, cw), 1) + pid * block_cols
    row = jax.lax.broadcasted_iota(jnp.int32, (rows, cw), 0)
    flat = (row * vocab + col).astype(jnp.uint32)

    bits = _threefry_bits(flat)
    mant = (bits >> jnp.uint32(9)) | jnp.uint32(0x3F800000)
    f = jax.lax.bitcast_convert_type(mant, jnp.float32) - jnp.float32(1.0)
    u = jnp.maximum(jnp.float32(_TINY), f + jnp.float32(_TINY))
    g = -jnp.log(-jnp.log(u))

    pert = blk / jnp.float32(_TEMPERATURE) + g
    pert = jnp.where(col < vocab, pert, -jnp.inf)

    bm = jnp.max(pert, axis=1, keepdims=True)
    cand = jnp.where(pert == bm, col, jnp.int32(0x7FFFFFFF))
    bi = jnp.min(cand, axis=1, keepdims=True)

    @pl.when(pid == 0)
    def _():
        val_ref[:, 0:1] = bm
        idx_ref[:, 0:1] = bi

    @pl.when(pid != 0)
    def _():
        prev_v = val_ref[:, 0:1]
        upd = bm > prev_v
        val_ref[:, 0:1] = jnp.where(upd, bm, prev_v)
        idx_ref[:, 0:1] = jnp.where(upd, bi, idx_ref[:, 0:1])

    @pl.when(pid == nblk - 1)
    def _():
        o_ref[...] = idx_ref[:, 0:1]


@jax.jit
def kernel(logits):
    rows, vocab = logits.shape
    block_cols = 2048
    nblk = pl.cdiv(vocab, block_cols)
    out = pl.pallas_call(
        functools.partial(_scan_body, vocab=vocab, block_cols=block_cols),
        grid=(nblk,),
        in_specs=[pl.BlockSpec((rows, block_cols), lambda i: (0, i))],
        out_specs=pl.BlockSpec((rows, 1), lambda i: (0, 0)),
        out_shape=jax.ShapeDtypeStruct((rows, 1), jnp.int32),
        scratch_shapes=[
            pltpu.VMEM((rows, 128), jnp.float32),
            pltpu.VMEM((rows, 128), jnp.int32),
        ],
        compiler_params=pltpu.CompilerParams(
            dimension_semantics=("arbitrary",),
        ),
    )(logits)
    return out.reshape(rows)
